# Initial kernel scaffold; baseline (speedup 1.0000x reference)
import jax
import jax.numpy as jnp
from jax.experimental import pallas as pl


def kernel(x, table):
    raise NotImplementedError("write your pallas kernel here")


# speedup vs baseline: 1.1038x; 1.1038x over previous
import functools

import jax
import jax.numpy as jnp
from jax import lax
from jax.experimental import pallas as pl
from jax.experimental.pallas import tpu as pltpu
from jax.experimental.pallas import tpu_sc as plsc

BATCH = 16384
HIST = 50
EMBED = 32
N = BATCH * HIST


def _build_sc_gather():
    info = plsc.get_sparse_core_info()
    nw = info.num_cores * info.num_subcores
    n_per_w = N // nw
    chunk = 1600
    n_chunks = n_per_w // chunk
    mesh = plsc.VectorSubcoreMesh(core_axis_name="c", subcore_axis_name="s")

    @functools.partial(
        pl.kernel,
        mesh=mesh,
        out_type=jax.ShapeDtypeStruct((N, EMBED), jnp.float32),
        scratch_types=[
            pltpu.VMEM((chunk,), jnp.int32),
            pltpu.VMEM((chunk, EMBED), jnp.float32),
            pltpu.SemaphoreType.DMA,
        ],
        compiler_params=pltpu.CompilerParams(use_tc_tiling_on_sc=False),
    )
    def gather_kernel(idx_hbm, table_hbm, out_hbm, idx_v, rows_v, sem):
        wid = lax.axis_index("s") * info.num_cores + lax.axis_index("c")
        base = wid * n_per_w

        def body(i, carry):
            off = base + i * chunk
            pltpu.sync_copy(idx_hbm.at[pl.ds(off, chunk)], idx_v)
            pltpu.async_copy(table_hbm.at[idx_v], rows_v, sem).wait()
            pltpu.sync_copy(rows_v, out_hbm.at[pl.ds(off, chunk)])
            return carry

        lax.fori_loop(0, n_chunks, body, 0)

    return gather_kernel


_sc_gather = _build_sc_gather()


def kernel(x, table):
    idx = x.reshape(N).astype(jnp.int32)
    out = _sc_gather(idx, table)
    return out.reshape(BATCH, HIST, EMBED)


# speedup vs baseline: 1.1104x; 1.0060x over previous
import functools

import jax
import jax.numpy as jnp
from jax import lax
from jax.experimental import pallas as pl
from jax.experimental.pallas import tpu as pltpu
from jax.experimental.pallas import tpu_sc as plsc

BATCH = 16384
HIST = 50
EMBED = 32
N = BATCH * HIST


def _build_sc_gather():
    info = plsc.get_sparse_core_info()
    nw = info.num_cores * info.num_subcores
    n_per_w = N // nw
    chunk = 1600
    n_chunks = n_per_w // chunk
    mesh = plsc.VectorSubcoreMesh(core_axis_name="c", subcore_axis_name="s")

    @functools.partial(
        pl.kernel,
        mesh=mesh,
        out_type=jax.ShapeDtypeStruct((N, EMBED), jnp.float32),
        scratch_types=[
            pltpu.VMEM((n_per_w,), jnp.int32),
            pltpu.VMEM((2, chunk, EMBED), jnp.float32),
            pltpu.SemaphoreType.DMA,
            pltpu.SemaphoreType.DMA,
            pltpu.SemaphoreType.DMA,
            pltpu.SemaphoreType.DMA,
        ],
        compiler_params=pltpu.CompilerParams(use_tc_tiling_on_sc=False),
    )
    def gather_kernel(idx_hbm, table_hbm, out_hbm, idx_v, rows_v, g0, g1, s0, s1):
        wid = lax.axis_index("s") * info.num_cores + lax.axis_index("c")
        base = wid * n_per_w
        gsem = (g0, g1)
        ssem = (s0, s1)

        pltpu.sync_copy(idx_hbm.at[pl.ds(base, n_per_w)], idx_v)

        def gather(i):
            return pltpu.async_copy(
                table_hbm.at[idx_v.at[pl.ds(i * chunk, chunk)]],
                rows_v.at[i % 2],
                gsem[i % 2],
            )

        def store(i):
            return pltpu.async_copy(
                rows_v.at[i % 2],
                out_hbm.at[pl.ds(base + i * chunk, chunk)],
                ssem[i % 2],
            )

        gathers = [None] * n_chunks
        stores = [None] * n_chunks
        gathers[0] = gather(0)
        for i in range(n_chunks):
            gathers[i].wait()
            stores[i] = store(i)
            if i + 1 < n_chunks:
                if i >= 1:
                    stores[i - 1].wait()
                gathers[i + 1] = gather(i + 1)
        stores[n_chunks - 2].wait()
        stores[n_chunks - 1].wait()

    return gather_kernel


_sc_gather = _build_sc_gather()


def kernel(x, table):
    idx = x.reshape(N)
    out = _sc_gather(idx, table)
    return out.reshape(BATCH, HIST, EMBED)


# speedup vs baseline: 1.8029x; 1.6236x over previous
import functools

import jax
import jax.numpy as jnp
from jax import lax
from jax.experimental import pallas as pl
from jax.experimental.pallas import tpu as pltpu
from jax.experimental.pallas import tpu_sc as plsc

BATCH = 16384
HIST = 50
EMBED = 32
W = 8
P = 4


def _build_sc_gather():
    info = plsc.get_sparse_core_info()
    nw = info.num_cores * info.num_subcores
    rows_per_w = BATCH // nw
    n_rounds = rows_per_w // W
    mesh = plsc.VectorSubcoreMesh(core_axis_name="c", subcore_axis_name="s")

    @functools.partial(
        pl.kernel,
        mesh=mesh,
        out_type=jax.ShapeDtypeStruct((BATCH, HIST, EMBED), jnp.float32),
        scratch_types=[
            pltpu.VMEM((rows_per_w, HIST), jnp.int32),
            pltpu.VMEM((P, W, HIST, EMBED), jnp.float32),
            [pltpu.SemaphoreType.DMA] * P,
            [pltpu.SemaphoreType.DMA] * P,
        ],
        compiler_params=pltpu.CompilerParams(use_tc_tiling_on_sc=False),
    )
    def gather_kernel(x_hbm, table_hbm, out_hbm, idx_v, rows_v, gsem, ssem):
        wid = lax.axis_index("s") * info.num_cores + lax.axis_index("c")
        base = wid * rows_per_w

        pltpu.sync_copy(x_hbm.at[pl.ds(base, rows_per_w)], idx_v)

        def fire_gathers(r, p):
            for k in range(W):
                pltpu.async_copy(
                    table_hbm.at[idx_v.at[r * W + k]], rows_v.at[p, k], gsem[p]
                )

        def drain_gathers(r, p):
            for k in range(W):
                pltpu.make_async_copy(
                    table_hbm.at[idx_v.at[r * W + k]], rows_v.at[p, k], gsem[p]
                ).wait()

        def fire_stores(r, p):
            for k in range(W):
                pltpu.async_copy(
                    rows_v.at[p, k], out_hbm.at[base + r * W + k], ssem[p]
                )

        def drain_stores(r, p):
            for k in range(W):
                pltpu.make_async_copy(
                    rows_v.at[p, k], out_hbm.at[base + r * W + k], ssem[p]
                ).wait()

        fire_gathers(0, 0)
        for r in range(1, P):
            fire_gathers(r, r % P)
            drain_gathers(r - 1, (r - 1) % P)
            fire_stores(r - 1, (r - 1) % P)

        def body(rr, carry):
            for q in range(P):
                r = P * rr + q
                drain_stores(r - P, q)
                fire_gathers(r, q)
                drain_gathers(r - 1, (q - 1) % P)
                fire_stores(r - 1, (q - 1) % P)
            return carry

        lax.fori_loop(1, n_rounds // P, body, 0)

        drain_gathers(n_rounds - 1, (n_rounds - 1) % P)
        fire_stores(n_rounds - 1, (n_rounds - 1) % P)
        for r in range(n_rounds - P, n_rounds):
            drain_stores(r, r % P)

    return gather_kernel


_sc_gather = _build_sc_gather()


def kernel(x, table):
    return _sc_gather(x, table)
